# Initial kernel scaffold; baseline (speedup 1.0000x reference)
import functools

import jax
import jax.numpy as jnp
from jax import lax
from jax.experimental import pallas as pl
from jax.experimental.pallas import tpu as pltpu
from jax.experimental.pallas import tpu_sc as plsc

N_NODES = 10000
N_EDGES = 320000
D_IN = 128
D_OUT = 128
DP = 144

NC = 2
NS = 16
NW = NC * NS
E_PER_W = N_EDGES // NW
EDGE_B = 80
NB = E_PER_W // EDGE_B
ROWS_PER_TILE = N_NODES // NS

_sc_mesh = plsc.VectorSubcoreMesh(core_axis_name="c", subcore_axis_name="s")


@functools.partial(
    pl.kernel,
    out_type=jax.ShapeDtypeStruct((NC, N_NODES, DP), jnp.float32),
    mesh=_sc_mesh,
    scratch_types=[
        pltpu.VMEM((EDGE_B,), jnp.int32),
        pltpu.VMEM((EDGE_B,), jnp.int32),
        pltpu.VMEM((EDGE_B, DP), jnp.float32),
        pltpu.VMEM_SHARED((N_NODES, DP), jnp.float32),
        pltpu.SemaphoreType.DMA,
    ],
)
def _sc_scatter(xa_hbm, col_hbm, row_hbm, zeros_hbm, out_hbm,
                colv, rowv, rowsv, agg_sh, sem):
    cid = lax.axis_index("c")
    sid = lax.axis_index("s")
    r0 = sid * ROWS_PER_TILE
    pltpu.sync_copy(zeros_hbm, agg_sh.at[pl.ds(r0, ROWS_PER_TILE)])
    plsc.subcore_barrier()

    ebase = (cid * NS + sid) * E_PER_W

    def body(g, carry):
        base = ebase + g * EDGE_B
        pltpu.sync_copy(col_hbm.at[pl.ds(base, EDGE_B)], colv)
        pltpu.async_copy(xa_hbm.at[colv], rowsv, sem).wait()
        pltpu.sync_copy(row_hbm.at[pl.ds(base, EDGE_B)], rowv)
        pltpu.sync_copy(rowsv, agg_sh.at[rowv], add=True)
        return carry

    lax.fori_loop(0, NB, body, 0)
    plsc.subcore_barrier()
    pltpu.sync_copy(agg_sh.at[pl.ds(r0, ROWS_PER_TILE)],
                    out_hbm.at[cid, pl.ds(r0, ROWS_PER_TILE)])


_TC_R = 1000


def _tc_body(x_ref, p0_ref, p1_ref, wt_ref, b_ref, o_ref):
    s = p0_ref[0, :, :D_IN] + p1_ref[0, :, :D_IN]
    deg = p0_ref[0, :, D_IN:D_IN + 1] + p1_ref[0, :, D_IN:D_IN + 1]
    agg = s / jnp.maximum(deg, 1.0)
    out = jnp.dot(x_ref[...], wt_ref[:D_IN, :],
                  preferred_element_type=jnp.float32)
    out += jnp.dot(agg, wt_ref[D_IN:, :], preferred_element_type=jnp.float32)
    o_ref[...] = out + b_ref[...]


def kernel(x, edge_index, W, b):
    ei = edge_index.astype(jnp.int32)
    row = ei[0]
    col = ei[1]
    ones_pad = jnp.concatenate(
        [jnp.ones((N_NODES, 1), jnp.float32),
         jnp.zeros((N_NODES, DP - D_IN - 1), jnp.float32)], axis=1)
    xa = jnp.concatenate([x.astype(jnp.float32), ones_pad], axis=1)
    zeros = jnp.zeros((ROWS_PER_TILE, DP), jnp.float32)

    partials = _sc_scatter(xa, col, row, zeros)

    wt = W.T.astype(jnp.float32)
    b2 = b.reshape(1, D_OUT).astype(jnp.float32)
    grid = (N_NODES // _TC_R,)
    return pl.pallas_call(
        _tc_body,
        grid=grid,
        in_specs=[
            pl.BlockSpec((_TC_R, D_IN), lambda i: (i, 0)),
            pl.BlockSpec((1, _TC_R, DP), lambda i: (0, i, 0)),
            pl.BlockSpec((1, _TC_R, DP), lambda i: (1, i, 0)),
            pl.BlockSpec((2 * D_IN, D_OUT), lambda i: (0, 0)),
            pl.BlockSpec((1, D_OUT), lambda i: (0, 0)),
        ],
        out_specs=pl.BlockSpec((_TC_R, D_OUT), lambda i: (i, 0)),
        out_shape=jax.ShapeDtypeStruct((N_NODES, D_OUT), jnp.float32),
    )(x.astype(jnp.float32), partials, partials, wt, b2)


# speedup vs baseline: 4.3482x; 4.3482x over previous
import functools

import jax
import jax.numpy as jnp
from jax import lax
from jax.experimental import pallas as pl
from jax.experimental.pallas import tpu as pltpu
from jax.experimental.pallas import tpu_sc as plsc

N_NODES = 10000
N_EDGES = 320000
D_IN = 128
D_OUT = 128
DP = 144

NC = 2
NS = 16
NW = NC * NS
E_PER_W = N_EDGES // NW
EDGE_B = 80
NB = E_PER_W // EDGE_B
N_PAD = 10240
ROWS_PER_TILE = N_PAD // NS

@functools.cache
def _build_sc_scatter():
    mesh = plsc.VectorSubcoreMesh(core_axis_name="c", subcore_axis_name="s",
                                  num_cores=NC, num_subcores=NS)

    @functools.partial(
        pl.kernel,
        out_type=jax.ShapeDtypeStruct((NC, N_PAD, DP), jnp.float32),
        mesh=mesh,
        scratch_types=[
            pltpu.VMEM((EDGE_B,), jnp.int32),
            pltpu.VMEM((EDGE_B,), jnp.int32),
            pltpu.VMEM((EDGE_B, DP), jnp.float32),
            pltpu.VMEM_SHARED((N_PAD, DP), jnp.float32),
            pltpu.SemaphoreType.DMA,
        ],
        compiler_params=pltpu.CompilerParams(use_tc_tiling_on_sc=False),
    )
    def _sc_scatter(xa_hbm, col_hbm, row_hbm, zeros_hbm, out_hbm,
                    colv, rowv, rowsv, agg_sh, sem):
        cid = lax.axis_index("c")
        sid = lax.axis_index("s")
        r0 = sid * ROWS_PER_TILE
        pltpu.sync_copy(zeros_hbm, agg_sh.at[pl.ds(r0, ROWS_PER_TILE)])
        plsc.subcore_barrier()

        ebase = (cid * NS + sid) * E_PER_W

        def body(g, carry):
            base = ebase + g * EDGE_B
            pltpu.sync_copy(col_hbm.at[pl.ds(base, EDGE_B)], colv)
            pltpu.async_copy(xa_hbm.at[colv], rowsv, sem).wait()
            pltpu.sync_copy(row_hbm.at[pl.ds(base, EDGE_B)], rowv)
            pltpu.sync_copy(rowsv, agg_sh.at[rowv], add=True)
            return carry

        lax.fori_loop(0, NB, body, 0)
        plsc.subcore_barrier()
        pltpu.sync_copy(agg_sh.at[pl.ds(r0, ROWS_PER_TILE)],
                        out_hbm.at[cid, pl.ds(r0, ROWS_PER_TILE)])

    return _sc_scatter


_TC_R = 1000


def _tc_body(x_ref, p0_ref, p1_ref, wt_ref, b_ref, o_ref):
    s = p0_ref[0, :, :D_IN] + p1_ref[0, :, :D_IN]
    deg = p0_ref[0, :, D_IN:D_IN + 1] + p1_ref[0, :, D_IN:D_IN + 1]
    agg = s / jnp.maximum(deg, 1.0)
    out = jnp.dot(x_ref[...], wt_ref[:D_IN, :],
                  preferred_element_type=jnp.float32)
    out += jnp.dot(agg, wt_ref[D_IN:, :], preferred_element_type=jnp.float32)
    o_ref[...] = out + b_ref[...]


def kernel(x, edge_index, W, b):
    ei = edge_index.astype(jnp.int32)
    row = ei[0]
    col = ei[1]
    ones_pad = jnp.concatenate(
        [jnp.ones((N_NODES, 1), jnp.float32),
         jnp.zeros((N_NODES, DP - D_IN - 1), jnp.float32)], axis=1)
    xa = jnp.concatenate([x.astype(jnp.float32), ones_pad], axis=1)
    zeros = jnp.zeros((ROWS_PER_TILE, DP), jnp.float32)

    partials = _build_sc_scatter()(xa, col, row, zeros)

    wt = W.T.astype(jnp.float32)
    b2 = b.reshape(1, D_OUT).astype(jnp.float32)
    grid = (N_NODES // _TC_R,)
    return pl.pallas_call(
        _tc_body,
        grid=grid,
        in_specs=[
            pl.BlockSpec((_TC_R, D_IN), lambda i: (i, 0)),
            pl.BlockSpec((1, _TC_R, DP), lambda i: (0, i, 0)),
            pl.BlockSpec((1, _TC_R, DP), lambda i: (1, i, 0)),
            pl.BlockSpec((2 * D_IN, D_OUT), lambda i: (0, 0)),
            pl.BlockSpec((1, D_OUT), lambda i: (0, 0)),
        ],
        out_specs=pl.BlockSpec((_TC_R, D_OUT), lambda i: (i, 0)),
        out_shape=jax.ShapeDtypeStruct((N_NODES, D_OUT), jnp.float32),
    )(x.astype(jnp.float32), partials, partials, wt, b2)
